# baseline (speedup 1.0000x reference)
import functools

import jax
import jax.numpy as jnp
from jax import lax
from jax.experimental import pallas as pl
from jax.experimental.pallas import tpu as pltpu
from jax.experimental.pallas import tpu_sc as plsc

_LANES = 128


def _widen_body(embt_ref, out_ref):
    out_ref[:, : embt_ref.shape[0]] = jnp.transpose(embt_ref[...], (1, 0))


@functools.lru_cache(maxsize=None)
def _make_tc_widen(vocab, hidden, tile_r):
    grid = (vocab + tile_r - 1) // tile_r
    return pl.pallas_call(
        _widen_body,
        grid=(grid,),
        in_specs=[pl.BlockSpec((hidden, tile_r), lambda i: (0, i))],
        out_specs=pl.BlockSpec((tile_r, _LANES), lambda i: (i, 0)),
        out_shape=jax.ShapeDtypeStruct((vocab, _LANES), jnp.float32),
    )


@functools.lru_cache(maxsize=None)
def _make_sc_gather(vocab, batch):
    info = plsc.get_sparse_core_info()
    nw = info.num_cores * info.num_subcores
    assert batch % nw == 0 and (batch // nw) % 8 == 0
    b_per_w = batch // nw
    mesh = plsc.VectorSubcoreMesh(core_axis_name="c", subcore_axis_name="s")

    @functools.partial(
        pl.kernel,
        mesh=mesh,
        out_type=jax.ShapeDtypeStruct((batch, _LANES), jnp.float32),
        scratch_types=[
            pltpu.VMEM((b_per_w,), jnp.int32),
            pltpu.VMEM((b_per_w, _LANES), jnp.float32),
            pltpu.SemaphoreType.DMA,
        ],
        compiler_params=pltpu.CompilerParams(use_tc_tiling_on_sc=False),
    )
    def gather_k(table_hbm, idx_hbm, out_hbm, idx_v, rows_v, sem):
        wid = lax.axis_index("s") * info.num_cores + lax.axis_index("c")
        base = wid * b_per_w
        pltpu.sync_copy(idx_hbm.at[pl.ds(base, b_per_w)], idx_v)
        pltpu.async_copy(table_hbm.at[idx_v], rows_v, sem).wait()
        pltpu.sync_copy(rows_v, out_hbm.at[pl.ds(base, b_per_w)])

    return gather_k


def _proj_body(h_ref, wt_ref, b_ref, out_ref):
    h = h_ref[:, : wt_ref.shape[0]]
    acc = lax.dot_general(
        wt_ref[...], h,
        (((0,), (1,)), ((), ())),
        preferred_element_type=jnp.float32,
    )
    tv, b = acc.shape
    acc3 = acc.reshape(tv // _LANES, _LANES, b) + b_ref[...].reshape(
        tv // _LANES, _LANES, 1)
    out_ref[...] = acc3.reshape(tv, b)


@functools.lru_cache(maxsize=None)
def _make_tc_proj(vocab, hidden, batch, tile_v):
    grid = (vocab + tile_v - 1) // tile_v
    return pl.pallas_call(
        _proj_body,
        grid=(grid,),
        in_specs=[
            pl.BlockSpec((batch, _LANES), lambda i: (0, 0)),
            pl.BlockSpec((hidden, tile_v), lambda i: (0, i)),
            pl.BlockSpec((tile_v // _LANES, _LANES), lambda i: (i, 0)),
        ],
        out_specs=pl.BlockSpec((tile_v, batch), lambda i: (i, 0)),
        out_shape=jax.ShapeDtypeStruct((vocab, batch), jnp.float32),
        compiler_params=pltpu.CompilerParams(
            vmem_limit_bytes=100 * 1024 * 1024,
        ),
    )


def kernel(x, emb_table, lin_w, lin_b):
    vocab, hidden = emb_table.shape
    batch = x.shape[0]
    tile_v = 2048
    grid = (vocab + tile_v - 1) // tile_v
    table_w = _make_tc_widen(vocab, hidden, 1024)(emb_table.T)
    h = _make_sc_gather(vocab, batch)(table_w, x.astype(jnp.int32))
    proj = _make_tc_proj(vocab, hidden, batch, tile_v)
    b2 = jnp.pad(lin_b, (0, grid * tile_v - vocab)).reshape(
        grid * tile_v // _LANES, _LANES)
    out_t = proj(h, lin_w.T, b2)
    return out_t.T
